# Initial kernel scaffold; baseline (speedup 1.0000x reference)
import jax
import jax.numpy as jnp
from jax.experimental import pallas as pl


def kernel(x, edge_index, batch, num_graphs, W_a1, b_a1, W_a2, b_a2, W_x1, b_x1, W_x2, b_x2, W_lin, b_lin):
    raise NotImplementedError("write your pallas kernel here")


# speedup vs baseline: 10.8150x; 10.8150x over previous
import functools

import jax
import jax.numpy as jnp
from jax import lax
from jax.experimental import pallas as pl
from jax.experimental.pallas import tpu as pltpu
from jax.experimental.pallas import tpu_sc as plsc

_N = 10000
_E = 320000
_G = 64
_NC = 2
_NS = 16
_NW = _NC * _NS
_CH = 128
_NCHUNK = 80
_EPT = _CH * _NCHUNK
_EPAD = _EPT * _NW
_NPAD = 10240
_RPT = _NPAD // _NS


def _sc_mesh():
    return plsc.VectorSubcoreMesh(core_axis_name="c", subcore_axis_name="s")


def _sc_degree(dstp, zeros_hist):
    @functools.partial(
        pl.kernel,
        out_type=jax.ShapeDtypeStruct((_NC * _NPAD,), jnp.float32),
        mesh=_sc_mesh(),
        scratch_types=[
            pltpu.VMEM((_EPT,), jnp.int32),
            pltpu.VMEM((_NPAD,), jnp.float32),
            pltpu.VMEM((_RPT,), jnp.float32),
            pltpu.VMEM((_RPT,), jnp.float32),
            pltpu.VMEM_SHARED((_NS, _NPAD), jnp.float32),
        ],
        compiler_params=pltpu.CompilerParams(needs_layout_passes=False),
    )
    def k(dst_hbm, zer_hbm, out_hbm, dix, hist, accb, tmpb, stage):
        c = lax.axis_index("c")
        s = lax.axis_index("s")
        w = c * _NS + s
        pltpu.sync_copy(zer_hbm, hist)
        pltpu.sync_copy(dst_hbm.at[pl.ds(w * _EPT, _EPT)], dix)
        ones = jnp.ones((16,), jnp.float32)

        def body(i, carry):
            idx = dix[pl.ds(i * 16, 16)]
            plsc.addupdate_scatter(hist, [idx], ones)
            return carry

        lax.fori_loop(0, _EPT // 16, body, 0)
        pltpu.sync_copy(hist, stage.at[s])
        plsc.subcore_barrier()

        pltpu.sync_copy(stage.at[0, pl.ds(s * _RPT, _RPT)], accb)
        for j in range(1, _NS):
            pltpu.sync_copy(stage.at[j, pl.ds(s * _RPT, _RPT)], tmpb)

            def addb(i, carry):
                sl = pl.ds(i * 16, 16)
                accb[sl] = accb[sl] + tmpb[sl]
                return carry

            lax.fori_loop(0, _RPT // 16, addb, 0)
        pltpu.sync_copy(accb, out_hbm.at[pl.ds(c * _NPAD + s * _RPT, _RPT)])

    return k(dstp, zeros_hist)


def _sc_prop(m, srcp, dstp, zeros_rows):
    @functools.partial(
        pl.kernel,
        out_type=jax.ShapeDtypeStruct((_NC * _NPAD, 128), jnp.float32),
        mesh=_sc_mesh(),
        scratch_types=[
            pltpu.VMEM((_CH,), jnp.int32),
            pltpu.VMEM((_CH,), jnp.int32),
            pltpu.VMEM((_CH, 128), jnp.float32),
            pltpu.VMEM_SHARED((_NPAD, 128), jnp.float32),
            pltpu.SemaphoreType.DMA,
        ],
    )
    def k(m_hbm, src_hbm, dst_hbm, zer_hbm, out_hbm, six, dix, rows, acc, sem):
        c = lax.axis_index("c")
        s = lax.axis_index("s")
        pltpu.sync_copy(zer_hbm, acc.at[pl.ds(s * _RPT, _RPT)])
        plsc.subcore_barrier()
        base = (c * _NS + s) * _EPT

        def body(i, carry):
            off = base + i * _CH
            pltpu.sync_copy(src_hbm.at[pl.ds(off, _CH)], six)
            pltpu.sync_copy(dst_hbm.at[pl.ds(off, _CH)], dix)
            pltpu.async_copy(m_hbm.at[six], rows, sem).wait()
            pltpu.sync_copy(rows, acc.at[dix], add=True)
            return carry

        lax.fori_loop(0, _NCHUNK, body, 0)
        plsc.subcore_barrier()
        pltpu.sync_copy(acc.at[pl.ds(s * _RPT, _RPT)],
                        out_hbm.at[pl.ds(c * _NPAD + s * _RPT, _RPT)])

    return k(m, srcp, dstp, zeros_rows)


def _tc_prep(x, deg0, deg1):
    def body(x_ref, d0_ref, d1_ref, m0_ref, dinv_ref):
        deg = d0_ref[...] + d1_ref[...] + 1.0
        dinv = 1.0 / jnp.sqrt(deg)
        dinv_ref[...] = dinv
        m0_ref[...] = x_ref[...] * dinv

    return pl.pallas_call(
        body,
        out_shape=(
            jax.ShapeDtypeStruct((_N, 128), jnp.float32),
            jax.ShapeDtypeStruct((_N, 1), jnp.float32),
        ),
    )(x, deg0, deg1)


def _tc_mid(p0, p1, m0, dinv, W_a1, b_a1, W_x1, b_x1, W_a2, W_x2):
    def body(p0_ref, p1_ref, m0_ref, dinv_ref, wa1, ba1, wx1, bx1, wa2, wx2,
             m1_ref):
        P = (p0_ref[...] + p1_ref[...] + m0_ref[...]) * dinv_ref[...]
        a1 = jnp.maximum(
            jnp.dot(P, wa1[...], preferred_element_type=jnp.float32) + ba1[...],
            0.0)
        x1 = jnp.maximum(
            jnp.dot(P, wx1[...], preferred_element_type=jnp.float32) + bx1[...],
            0.0)
        ha = jnp.dot(a1, wa2[...], preferred_element_type=jnp.float32)
        hx = jnp.dot(x1, wx2[...], preferred_element_type=jnp.float32)
        m1_ref[...] = jnp.concatenate([ha, hx], axis=1) * dinv_ref[...]

    nb = 10
    rb = _N // nb
    return pl.pallas_call(
        body,
        grid=(nb,),
        in_specs=[
            pl.BlockSpec((rb, 128), lambda i: (i, 0)),
            pl.BlockSpec((rb, 128), lambda i: (i, 0)),
            pl.BlockSpec((rb, 128), lambda i: (i, 0)),
            pl.BlockSpec((rb, 1), lambda i: (i, 0)),
            pl.BlockSpec((128, 128), lambda i: (0, 0)),
            pl.BlockSpec((1, 128), lambda i: (0, 0)),
            pl.BlockSpec((128, 128), lambda i: (0, 0)),
            pl.BlockSpec((1, 128), lambda i: (0, 0)),
            pl.BlockSpec((128, 64), lambda i: (0, 0)),
            pl.BlockSpec((128, 64), lambda i: (0, 0)),
        ],
        out_specs=pl.BlockSpec((rb, 128), lambda i: (i, 0)),
        out_shape=jax.ShapeDtypeStruct((_N, 128), jnp.float32),
    )(p0, p1, m0, dinv, W_a1, b_a1, W_x1, b_x1, W_a2, W_x2)


def _tc_pre(q0, q1, m1, dinv, batch_col, b_a2, b_x2):
    def body(q0_ref, q1_ref, m1_ref, dinv_ref, b_ref, ba2, bx2,
             s_ref, x2_ref, oh_ref):
        q = (q0_ref[...] + q1_ref[...] + m1_ref[...]) * dinv_ref[...]
        s_ref[...] = q[:, :64] + ba2[...]
        x2_ref[...] = jnp.maximum(q[:, 64:] + bx2[...], 0.0)
        oh_ref[...] = (b_ref[...] == lax.broadcasted_iota(jnp.int32, (1, _G), 1)
                       ).astype(jnp.float32)

    return pl.pallas_call(
        body,
        out_shape=(
            jax.ShapeDtypeStruct((_N, 64), jnp.float32),
            jax.ShapeDtypeStruct((_N, 64), jnp.float32),
            jax.ShapeDtypeStruct((_N, _G), jnp.float32),
        ),
    )(q0, q1, m1, dinv, batch_col, b_a2, b_x2)


def _tc_segmax(s, batch_col):
    def body(s_ref, b_ref, m_ref):
        gb = pl.program_id(0) * 8
        b = b_ref[...]
        s = s_ref[...]
        rows = [jnp.max(jnp.where(b == gb + j, s, -jnp.inf), axis=0,
                        keepdims=True) for j in range(8)]
        m_ref[...] = jnp.concatenate(rows, axis=0)

    return pl.pallas_call(
        body,
        grid=(_G // 8,),
        in_specs=[
            pl.BlockSpec((_N, 64), lambda i: (0, 0)),
            pl.BlockSpec((_N, 1), lambda i: (0, 0)),
        ],
        out_specs=pl.BlockSpec((8, 64), lambda i: (i, 0)),
        out_shape=jax.ShapeDtypeStruct((_G, 64), jnp.float32),
    )(s, batch_col)


def _tc_softmax_norm(s, m, onehot):
    def body(s_ref, m_ref, oh_ref, a2_ref):
        m = m_ref[...]
        m = jnp.where(jnp.isfinite(m), m, 0.0)
        onehot = oh_ref[...]
        mb = jnp.dot(onehot, m, preferred_element_type=jnp.float32)
        e = jnp.exp(s_ref[...] - mb)
        S = lax.dot_general(onehot, e, (((0,), (0,)), ((), ())),
                            preferred_element_type=jnp.float32)
        Sb = jnp.dot(onehot, S, preferred_element_type=jnp.float32)
        a2_ref[...] = e / (Sb + 1e-16)

    return pl.pallas_call(
        body,
        out_shape=jax.ShapeDtypeStruct((_N, 64), jnp.float32),
    )(s, m, onehot)


def _tc_outer(a2, x2, batch_col):
    def body(a2_ref, x2_ref, b_ref, t_ref):
        g = pl.program_id(0)
        mask = (b_ref[...] == g).astype(jnp.float32)
        Ag = a2_ref[...] * mask
        t_ref[0] = lax.dot_general(Ag, x2_ref[...], (((0,), (0,)), ((), ())),
                                   preferred_element_type=jnp.float32)

    return pl.pallas_call(
        body,
        grid=(_G,),
        in_specs=[
            pl.BlockSpec((_N, 64), lambda g: (0, 0)),
            pl.BlockSpec((_N, 64), lambda g: (0, 0)),
            pl.BlockSpec((_N, 1), lambda g: (0, 0)),
        ],
        out_specs=pl.BlockSpec((1, 64, 64), lambda g: (g, 0, 0)),
        out_shape=jax.ShapeDtypeStruct((_G, 64, 64), jnp.float32),
    )(a2, x2, batch_col)


def _tc_head(tflat, W_lin, b_lin):
    def body(t_ref, wl, bl, out_ref):
        logits = jnp.dot(t_ref[...], wl[...],
                         preferred_element_type=jnp.float32) + bl[...]
        lm = jnp.max(logits, axis=1, keepdims=True)
        le = jnp.exp(logits - lm)
        out_ref[...] = le / jnp.sum(le, axis=1, keepdims=True)

    return pl.pallas_call(
        body,
        out_shape=jax.ShapeDtypeStruct((_G, 10), jnp.float32),
    )(tflat, W_lin, b_lin)


def kernel(x, edge_index, batch, num_graphs, W_a1, b_a1, W_a2, b_a2,
           W_x1, b_x1, W_x2, b_x2, W_lin, b_lin):
    src = edge_index[0]
    dst = edge_index[1]
    npad_e = _EPAD - _E
    srcp = jnp.concatenate([src, jnp.zeros((npad_e,), jnp.int32)])
    dstp = jnp.concatenate([dst, jnp.full((npad_e,), _N, jnp.int32)])

    zeros_hist = jnp.zeros((_NPAD,), jnp.float32)
    zeros_rows = jnp.zeros((_RPT, 128), jnp.float32)

    degf = _sc_degree(dstp, zeros_hist)
    deg0 = degf[:_N].reshape(_N, 1)
    deg1 = degf[_NPAD:_NPAD + _N].reshape(_N, 1)

    m0, dinv = _tc_prep(x, deg0, deg1)

    p = _sc_prop(m0, srcp, dstp, zeros_rows)
    m1 = _tc_mid(p[:_N], p[_NPAD:_NPAD + _N], m0, dinv,
                 W_a1, b_a1.reshape(1, 128), W_x1, b_x1.reshape(1, 128),
                 W_a2, W_x2)

    q = _sc_prop(m1, srcp, dstp, zeros_rows)
    batch_col = batch.reshape(_N, 1)
    s, x2, onehot = _tc_pre(q[:_N], q[_NPAD:_NPAD + _N], m1, dinv,
                            batch_col, b_a2.reshape(1, 64),
                            b_x2.reshape(1, 64))
    m = _tc_segmax(s, batch_col)
    a2 = _tc_softmax_norm(s, m, onehot)
    tall = _tc_outer(a2, x2, batch_col)
    out = _tc_head(tall.reshape(_G, 64 * 64), W_lin, b_lin.reshape(1, 10))
    return out


# speedup vs baseline: 13.6797x; 1.2649x over previous
import functools

import jax
import jax.numpy as jnp
from jax import lax
from jax.experimental import pallas as pl
from jax.experimental.pallas import tpu as pltpu
from jax.experimental.pallas import tpu_sc as plsc

_N = 10000
_E = 320000
_G = 64
_NC = 2
_NS = 16
_NW = _NC * _NS
_CH = 128
_NCHUNK = 80
_HCHUNK = _NCHUNK // 2
_EPT = _CH * _NCHUNK
_EPAD = _EPT * _NW
_NPAD = 10240
_RPT = _NPAD // _NS


def _sc_mesh():
    return plsc.VectorSubcoreMesh(core_axis_name="c", subcore_axis_name="s")


def _sc_degree(dstp, zeros_hist):
    @functools.partial(
        pl.kernel,
        out_type=jax.ShapeDtypeStruct((_NC * _NPAD,), jnp.float32),
        mesh=_sc_mesh(),
        scratch_types=[
            pltpu.VMEM((_EPT,), jnp.int32),
            pltpu.VMEM((_NPAD,), jnp.float32),
            pltpu.VMEM((_RPT,), jnp.float32),
            pltpu.VMEM((_RPT,), jnp.float32),
            pltpu.VMEM_SHARED((_NS, _NPAD), jnp.float32),
        ],
        compiler_params=pltpu.CompilerParams(needs_layout_passes=False),
    )
    def k(dst_hbm, zer_hbm, out_hbm, dix, hist, accb, tmpb, stage):
        c = lax.axis_index("c")
        s = lax.axis_index("s")
        w = c * _NS + s
        pltpu.sync_copy(zer_hbm, hist)
        pltpu.sync_copy(dst_hbm.at[pl.ds(w * _EPT, _EPT)], dix)
        ones = jnp.ones((16,), jnp.float32)

        def body(i, carry):
            idx = dix[pl.ds(i * 16, 16)]
            plsc.addupdate_scatter(hist, [idx], ones)
            return carry

        lax.fori_loop(0, _EPT // 16, body, 0)
        pltpu.sync_copy(hist, stage.at[s])
        plsc.subcore_barrier()

        pltpu.sync_copy(stage.at[0, pl.ds(s * _RPT, _RPT)], accb)
        for j in range(1, _NS):
            pltpu.sync_copy(stage.at[j, pl.ds(s * _RPT, _RPT)], tmpb)

            def addb(i, carry):
                sl = pl.ds(i * 16, 16)
                accb[sl] = accb[sl] + tmpb[sl]
                return carry

            lax.fori_loop(0, _RPT // 16, addb, 0)
        pltpu.sync_copy(accb, out_hbm.at[pl.ds(c * _NPAD + s * _RPT, _RPT)])

    return k(dstp, zeros_hist)


def _sc_prop(m, src2d, dst2d, zeros_rows):
    @functools.partial(
        pl.kernel,
        out_type=jax.ShapeDtypeStruct((_NC * _NPAD, 128), jnp.float32),
        mesh=_sc_mesh(),
        scratch_types=[
            pltpu.VMEM((_HCHUNK, _CH), jnp.int32),
            pltpu.VMEM((_HCHUNK, _CH), jnp.int32),
            pltpu.VMEM((_CH, 128), jnp.float32),
            pltpu.VMEM((_CH, 128), jnp.float32),
            pltpu.VMEM_SHARED((_NPAD, 128), jnp.float32),
            pltpu.SemaphoreType.DMA,
            pltpu.SemaphoreType.DMA,
        ],
    )
    def k(m_hbm, src_hbm, dst_hbm, zer_hbm, out_hbm,
          six2, dix2, rows0, rows1, acc, gsem, ssem):
        c = lax.axis_index("c")
        s = lax.axis_index("s")
        pltpu.sync_copy(zer_hbm, acc.at[pl.ds(s * _RPT, _RPT)])
        w = c * _NS + s
        plsc.subcore_barrier()

        bufs = (rows0, rows1)

        def fire_gather(g, buf):
            pltpu.async_copy(m_hbm.at[six2.at[g]], buf, gsem)

        def drain_gather(buf):
            pltpu.make_async_copy(m_hbm.at[pl.ds(0, _CH)], buf, gsem).wait()

        def fire_scatter(g, buf):
            pltpu.async_copy(buf, acc.at[dix2.at[g]], ssem, add=True)

        def drain_scatter(buf):
            pltpu.make_async_copy(buf, acc.at[pl.ds(0, _CH)], ssem).wait()

        def run_half(h):
            row0 = w * _NCHUNK + h * _HCHUNK
            pltpu.sync_copy(src_hbm.at[pl.ds(row0, _HCHUNK)], six2)
            pltpu.sync_copy(dst_hbm.at[pl.ds(row0, _HCHUNK)], dix2)
            fire_gather(0, bufs[0])

            def body(tt, carry):
                for ph in (0, 1):
                    g = 2 * tt + ph
                    buf = bufs[ph]
                    drain_gather(buf)
                    fire_scatter(g, buf)

                    @pl.when(g >= 1)
                    def _():
                        drain_scatter(bufs[1 - ph])

                    @pl.when(g + 1 < _HCHUNK)
                    def _():
                        fire_gather(g + 1, bufs[1 - ph])
                return carry

            lax.fori_loop(0, _HCHUNK // 2, body, 0)
            drain_scatter(bufs[1])

        run_half(0)
        run_half(1)
        plsc.subcore_barrier()
        pltpu.sync_copy(acc.at[pl.ds(s * _RPT, _RPT)],
                        out_hbm.at[pl.ds(c * _NPAD + s * _RPT, _RPT)])

    return k(m, src2d, dst2d, zeros_rows)


def _tc_prep(x, deg0, deg1):
    def body(x_ref, d0_ref, d1_ref, m0_ref, dinv_ref):
        deg = d0_ref[...] + d1_ref[...] + 1.0
        dinv = 1.0 / jnp.sqrt(deg)
        dinv_ref[...] = dinv
        m0_ref[...] = x_ref[...] * dinv

    return pl.pallas_call(
        body,
        out_shape=(
            jax.ShapeDtypeStruct((_N, 128), jnp.float32),
            jax.ShapeDtypeStruct((_N, 1), jnp.float32),
        ),
    )(x, deg0, deg1)


def _tc_mid(p0, p1, m0, dinv, W_a1, b_a1, W_x1, b_x1, W_a2, W_x2):
    def body(p0_ref, p1_ref, m0_ref, dinv_ref, wa1, ba1, wx1, bx1, wa2, wx2,
             m1_ref):
        P = (p0_ref[...] + p1_ref[...] + m0_ref[...]) * dinv_ref[...]
        a1 = jnp.maximum(
            jnp.dot(P, wa1[...], preferred_element_type=jnp.float32) + ba1[...],
            0.0)
        x1 = jnp.maximum(
            jnp.dot(P, wx1[...], preferred_element_type=jnp.float32) + bx1[...],
            0.0)
        ha = jnp.dot(a1, wa2[...], preferred_element_type=jnp.float32)
        hx = jnp.dot(x1, wx2[...], preferred_element_type=jnp.float32)
        m1_ref[...] = jnp.concatenate([ha, hx], axis=1) * dinv_ref[...]

    nb = 10
    rb = _N // nb
    return pl.pallas_call(
        body,
        grid=(nb,),
        in_specs=[
            pl.BlockSpec((rb, 128), lambda i: (i, 0)),
            pl.BlockSpec((rb, 128), lambda i: (i, 0)),
            pl.BlockSpec((rb, 128), lambda i: (i, 0)),
            pl.BlockSpec((rb, 1), lambda i: (i, 0)),
            pl.BlockSpec((128, 128), lambda i: (0, 0)),
            pl.BlockSpec((1, 128), lambda i: (0, 0)),
            pl.BlockSpec((128, 128), lambda i: (0, 0)),
            pl.BlockSpec((1, 128), lambda i: (0, 0)),
            pl.BlockSpec((128, 64), lambda i: (0, 0)),
            pl.BlockSpec((128, 64), lambda i: (0, 0)),
        ],
        out_specs=pl.BlockSpec((rb, 128), lambda i: (i, 0)),
        out_shape=jax.ShapeDtypeStruct((_N, 128), jnp.float32),
    )(p0, p1, m0, dinv, W_a1, b_a1, W_x1, b_x1, W_a2, W_x2)


def _tc_pre(q0, q1, m1, dinv, batch_col, b_a2, b_x2):
    def body(q0_ref, q1_ref, m1_ref, dinv_ref, b_ref, ba2, bx2,
             s_ref, x2_ref, oh_ref):
        q = (q0_ref[...] + q1_ref[...] + m1_ref[...]) * dinv_ref[...]
        s_ref[...] = q[:, :64] + ba2[...]
        x2_ref[...] = jnp.maximum(q[:, 64:] + bx2[...], 0.0)
        oh_ref[...] = (b_ref[...] == lax.broadcasted_iota(jnp.int32, (1, _G), 1)
                       ).astype(jnp.float32)

    return pl.pallas_call(
        body,
        out_shape=(
            jax.ShapeDtypeStruct((_N, 64), jnp.float32),
            jax.ShapeDtypeStruct((_N, 64), jnp.float32),
            jax.ShapeDtypeStruct((_N, _G), jnp.float32),
        ),
    )(q0, q1, m1, dinv, batch_col, b_a2, b_x2)


def _tc_segmax(s, batch_col):
    def body(s_ref, b_ref, m_ref):
        gb = pl.program_id(0) * 8
        b = b_ref[...]
        s = s_ref[...]
        rows = [jnp.max(jnp.where(b == gb + j, s, -jnp.inf), axis=0,
                        keepdims=True) for j in range(8)]
        m_ref[...] = jnp.concatenate(rows, axis=0)

    return pl.pallas_call(
        body,
        grid=(_G // 8,),
        in_specs=[
            pl.BlockSpec((_N, 64), lambda i: (0, 0)),
            pl.BlockSpec((_N, 1), lambda i: (0, 0)),
        ],
        out_specs=pl.BlockSpec((8, 64), lambda i: (i, 0)),
        out_shape=jax.ShapeDtypeStruct((_G, 64), jnp.float32),
    )(s, batch_col)


def _tc_softmax_norm(s, m, onehot):
    def body(s_ref, m_ref, oh_ref, a2_ref):
        m = m_ref[...]
        m = jnp.where(jnp.isfinite(m), m, 0.0)
        onehot = oh_ref[...]
        mb = jnp.dot(onehot, m, preferred_element_type=jnp.float32)
        e = jnp.exp(s_ref[...] - mb)
        S = lax.dot_general(onehot, e, (((0,), (0,)), ((), ())),
                            preferred_element_type=jnp.float32)
        Sb = jnp.dot(onehot, S, preferred_element_type=jnp.float32)
        a2_ref[...] = e / (Sb + 1e-16)

    return pl.pallas_call(
        body,
        out_shape=jax.ShapeDtypeStruct((_N, 64), jnp.float32),
    )(s, m, onehot)


def _tc_outer(a2, x2, batch_col):
    def body(a2_ref, x2_ref, b_ref, t_ref):
        g = pl.program_id(0)
        mask = (b_ref[...] == g).astype(jnp.float32)
        Ag = a2_ref[...] * mask
        t_ref[0] = lax.dot_general(Ag, x2_ref[...], (((0,), (0,)), ((), ())),
                                   preferred_element_type=jnp.float32)

    return pl.pallas_call(
        body,
        grid=(_G,),
        in_specs=[
            pl.BlockSpec((_N, 64), lambda g: (0, 0)),
            pl.BlockSpec((_N, 64), lambda g: (0, 0)),
            pl.BlockSpec((_N, 1), lambda g: (0, 0)),
        ],
        out_specs=pl.BlockSpec((1, 64, 64), lambda g: (g, 0, 0)),
        out_shape=jax.ShapeDtypeStruct((_G, 64, 64), jnp.float32),
    )(a2, x2, batch_col)


def _tc_head(tflat, W_lin, b_lin):
    def body(t_ref, wl, bl, out_ref):
        logits = jnp.dot(t_ref[...], wl[...],
                         preferred_element_type=jnp.float32) + bl[...]
        lm = jnp.max(logits, axis=1, keepdims=True)
        le = jnp.exp(logits - lm)
        out_ref[...] = le / jnp.sum(le, axis=1, keepdims=True)

    return pl.pallas_call(
        body,
        out_shape=jax.ShapeDtypeStruct((_G, 10), jnp.float32),
    )(tflat, W_lin, b_lin)


def kernel(x, edge_index, batch, num_graphs, W_a1, b_a1, W_a2, b_a2,
           W_x1, b_x1, W_x2, b_x2, W_lin, b_lin):
    src = edge_index[0]
    dst = edge_index[1]
    npad_e = _EPAD - _E
    srcp = jnp.concatenate([src, jnp.zeros((npad_e,), jnp.int32)])
    dstp = jnp.concatenate([dst, jnp.full((npad_e,), _N, jnp.int32)])
    src2d = srcp.reshape(_EPAD // _CH, _CH)
    dst2d = dstp.reshape(_EPAD // _CH, _CH)

    zeros_hist = jnp.zeros((_NPAD,), jnp.float32)
    zeros_rows = jnp.zeros((_RPT, 128), jnp.float32)

    degf = _sc_degree(dstp, zeros_hist)
    deg0 = degf[:_N].reshape(_N, 1)
    deg1 = degf[_NPAD:_NPAD + _N].reshape(_N, 1)

    m0, dinv = _tc_prep(x, deg0, deg1)

    p = _sc_prop(m0, src2d, dst2d, zeros_rows)
    m1 = _tc_mid(p[:_N], p[_NPAD:_NPAD + _N], m0, dinv,
                 W_a1, b_a1.reshape(1, 128), W_x1, b_x1.reshape(1, 128),
                 W_a2, W_x2)

    q = _sc_prop(m1, src2d, dst2d, zeros_rows)
    batch_col = batch.reshape(_N, 1)
    s, x2, onehot = _tc_pre(q[:_N], q[_NPAD:_NPAD + _N], m1, dinv,
                            batch_col, b_a2.reshape(1, 64),
                            b_x2.reshape(1, 64))
    m = _tc_segmax(s, batch_col)
    a2 = _tc_softmax_norm(s, m, onehot)
    tall = _tc_outer(a2, x2, batch_col)
    out = _tc_head(tall.reshape(_G, 64 * 64), W_lin, b_lin.reshape(1, 10))
    return out


# speedup vs baseline: 14.4508x; 1.0564x over previous
import functools

import jax
import jax.numpy as jnp
from jax import lax
from jax.experimental import pallas as pl
from jax.experimental.pallas import tpu as pltpu
from jax.experimental.pallas import tpu_sc as plsc

_N = 10000
_E = 320000
_G = 64
_NC = 2
_NS = 16
_NW = _NC * _NS
_CH = 128
_NCHUNK = 80
_NCH0, _NCH1 = 120, 40
_SEG0, _SEG1 = 24, 8
_NSEG = 5
_EPT = _CH * _NCHUNK
_EPAD = _EPT * _NW
_NPAD = 10240
_RPT = _NPAD // _NS


def _sc_mesh():
    return plsc.VectorSubcoreMesh(core_axis_name="c", subcore_axis_name="s")


def _sc_degree(dstp, zeros_hist):
    @functools.partial(
        pl.kernel,
        out_type=jax.ShapeDtypeStruct((_NC * _NPAD,), jnp.float32),
        mesh=_sc_mesh(),
        scratch_types=[
            pltpu.VMEM((_EPT,), jnp.int32),
            pltpu.VMEM((_NPAD,), jnp.float32),
            pltpu.VMEM((_RPT,), jnp.float32),
            pltpu.VMEM((_RPT,), jnp.float32),
            pltpu.VMEM_SHARED((_NS, _NPAD), jnp.float32),
        ],
        compiler_params=pltpu.CompilerParams(needs_layout_passes=False),
    )
    def k(dst_hbm, zer_hbm, out_hbm, dix, hist, accb, tmpb, stage):
        c = lax.axis_index("c")
        s = lax.axis_index("s")
        w = c * _NS + s
        pltpu.sync_copy(zer_hbm, hist)
        pltpu.sync_copy(dst_hbm.at[pl.ds(w * _EPT, _EPT)], dix)
        ones = jnp.ones((16,), jnp.float32)

        def body(i, carry):
            idx = dix[pl.ds(i * 16, 16)]
            plsc.addupdate_scatter(hist, [idx], ones)
            return carry

        lax.fori_loop(0, _EPT // 16, body, 0)
        pltpu.sync_copy(hist, stage.at[s])
        plsc.subcore_barrier()

        pltpu.sync_copy(stage.at[0, pl.ds(s * _RPT, _RPT)], accb)
        for j in range(1, _NS):
            pltpu.sync_copy(stage.at[j, pl.ds(s * _RPT, _RPT)], tmpb)

            def addb(i, carry):
                sl = pl.ds(i * 16, 16)
                accb[sl] = accb[sl] + tmpb[sl]
                return carry

            lax.fori_loop(0, _RPT // 16, addb, 0)
        pltpu.sync_copy(accb, out_hbm.at[pl.ds(c * _NPAD + s * _RPT, _RPT)])

    return k(dstp, zeros_hist)


def _sc_prop(m, src2d, dst2d, zeros_rows):
    @functools.partial(
        pl.kernel,
        out_type=jax.ShapeDtypeStruct((_NC * _NPAD, 128), jnp.float32),
        mesh=_sc_mesh(),
        scratch_types=[
            pltpu.VMEM((_SEG0, _CH), jnp.int32),
            pltpu.VMEM((_SEG0, _CH), jnp.int32),
            pltpu.VMEM((_CH, 128), jnp.float32),
            pltpu.VMEM((_CH, 128), jnp.float32),
            pltpu.VMEM_SHARED((_NPAD, 128), jnp.float32),
            pltpu.SemaphoreType.DMA,
            pltpu.SemaphoreType.DMA,
        ],
    )
    def k(m_hbm, src_hbm, dst_hbm, zer_hbm, out_hbm,
          six2, dix2, rows0, rows1, acc, gsem, ssem):
        c = lax.axis_index("c")
        s = lax.axis_index("s")
        pltpu.sync_copy(zer_hbm, acc.at[pl.ds(s * _RPT, _RPT)])
        plsc.subcore_barrier()

        bufs = (rows0, rows1)

        def fire_gather(g, buf):
            pltpu.async_copy(m_hbm.at[six2.at[g]], buf, gsem)

        def drain_gather(buf):
            pltpu.make_async_copy(m_hbm.at[pl.ds(0, _CH)], buf, gsem).wait()

        def fire_scatter(g, buf):
            pltpu.async_copy(buf, acc.at[dix2.at[g]], ssem, add=True)

        def drain_scatter(buf):
            pltpu.make_async_copy(buf, acc.at[pl.ds(0, _CH)], ssem).wait()

        def run_seg(row0, nch):
            pltpu.sync_copy(src_hbm.at[pl.ds(row0, nch)],
                            six2.at[pl.ds(0, nch)])
            pltpu.sync_copy(dst_hbm.at[pl.ds(row0, nch)],
                            dix2.at[pl.ds(0, nch)])
            fire_gather(0, bufs[0])

            def body(tt, carry):
                for ph in (0, 1):
                    g = 2 * tt + ph
                    buf = bufs[ph]
                    drain_gather(buf)
                    fire_scatter(g, buf)

                    @pl.when(g >= 1)
                    def _():
                        drain_scatter(bufs[1 - ph])

                    @pl.when(g + 1 < nch)
                    def _():
                        fire_gather(g + 1, bufs[1 - ph])
                return carry

            lax.fori_loop(0, nch // 2, body, 0)
            drain_scatter(bufs[1])

        @pl.when(c == 0)
        def _():
            for seg in range(_NSEG):
                run_seg(s * _NCH0 + seg * _SEG0, _SEG0)

        @pl.when(c == 1)
        def _():
            base = _NS * _NCH0
            for seg in range(_NSEG):
                run_seg(base + s * _NCH1 + seg * _SEG1, _SEG1)

        plsc.subcore_barrier()
        pltpu.sync_copy(acc.at[pl.ds(s * _RPT, _RPT)],
                        out_hbm.at[pl.ds(c * _NPAD + s * _RPT, _RPT)])

    return k(m, src2d, dst2d, zeros_rows)


def _tc_prep(x, deg0, deg1):
    def body(x_ref, d0_ref, d1_ref, m0_ref, dinv_ref):
        deg = d0_ref[...] + d1_ref[...] + 1.0
        dinv = 1.0 / jnp.sqrt(deg)
        dinv_ref[...] = dinv
        m0_ref[...] = x_ref[...] * dinv

    return pl.pallas_call(
        body,
        out_shape=(
            jax.ShapeDtypeStruct((_N, 128), jnp.float32),
            jax.ShapeDtypeStruct((_N, 1), jnp.float32),
        ),
    )(x, deg0, deg1)


def _tc_mid(p0, p1, m0, dinv, W_a1, b_a1, W_x1, b_x1, W_a2, W_x2):
    def body(p0_ref, p1_ref, m0_ref, dinv_ref, wa1, ba1, wx1, bx1, wa2, wx2,
             m1_ref):
        P = (p0_ref[...] + p1_ref[...] + m0_ref[...]) * dinv_ref[...]
        a1 = jnp.maximum(
            jnp.dot(P, wa1[...], preferred_element_type=jnp.float32) + ba1[...],
            0.0)
        x1 = jnp.maximum(
            jnp.dot(P, wx1[...], preferred_element_type=jnp.float32) + bx1[...],
            0.0)
        ha = jnp.dot(a1, wa2[...], preferred_element_type=jnp.float32)
        hx = jnp.dot(x1, wx2[...], preferred_element_type=jnp.float32)
        m1_ref[...] = jnp.concatenate([ha, hx], axis=1) * dinv_ref[...]

    nb = 10
    rb = _N // nb
    return pl.pallas_call(
        body,
        grid=(nb,),
        in_specs=[
            pl.BlockSpec((rb, 128), lambda i: (i, 0)),
            pl.BlockSpec((rb, 128), lambda i: (i, 0)),
            pl.BlockSpec((rb, 128), lambda i: (i, 0)),
            pl.BlockSpec((rb, 1), lambda i: (i, 0)),
            pl.BlockSpec((128, 128), lambda i: (0, 0)),
            pl.BlockSpec((1, 128), lambda i: (0, 0)),
            pl.BlockSpec((128, 128), lambda i: (0, 0)),
            pl.BlockSpec((1, 128), lambda i: (0, 0)),
            pl.BlockSpec((128, 64), lambda i: (0, 0)),
            pl.BlockSpec((128, 64), lambda i: (0, 0)),
        ],
        out_specs=pl.BlockSpec((rb, 128), lambda i: (i, 0)),
        out_shape=jax.ShapeDtypeStruct((_N, 128), jnp.float32),
    )(p0, p1, m0, dinv, W_a1, b_a1, W_x1, b_x1, W_a2, W_x2)


def _tc_pre(q0, q1, m1, dinv, batch_col, b_a2, b_x2):
    def body(q0_ref, q1_ref, m1_ref, dinv_ref, b_ref, ba2, bx2,
             s_ref, x2_ref, oh_ref):
        q = (q0_ref[...] + q1_ref[...] + m1_ref[...]) * dinv_ref[...]
        s_ref[...] = q[:, :64] + ba2[...]
        x2_ref[...] = jnp.maximum(q[:, 64:] + bx2[...], 0.0)
        oh_ref[...] = (b_ref[...] == lax.broadcasted_iota(jnp.int32, (1, _G), 1)
                       ).astype(jnp.float32)

    return pl.pallas_call(
        body,
        out_shape=(
            jax.ShapeDtypeStruct((_N, 64), jnp.float32),
            jax.ShapeDtypeStruct((_N, 64), jnp.float32),
            jax.ShapeDtypeStruct((_N, _G), jnp.float32),
        ),
    )(q0, q1, m1, dinv, batch_col, b_a2, b_x2)


def _tc_segmax(s, batch_col):
    def body(s_ref, b_ref, m_ref):
        gb = pl.program_id(0) * 8
        b = b_ref[...]
        s = s_ref[...]
        rows = [jnp.max(jnp.where(b == gb + j, s, -jnp.inf), axis=0,
                        keepdims=True) for j in range(8)]
        m_ref[...] = jnp.concatenate(rows, axis=0)

    return pl.pallas_call(
        body,
        grid=(_G // 8,),
        in_specs=[
            pl.BlockSpec((_N, 64), lambda i: (0, 0)),
            pl.BlockSpec((_N, 1), lambda i: (0, 0)),
        ],
        out_specs=pl.BlockSpec((8, 64), lambda i: (i, 0)),
        out_shape=jax.ShapeDtypeStruct((_G, 64), jnp.float32),
    )(s, batch_col)


def _tc_softmax_norm(s, m, onehot):
    def body(s_ref, m_ref, oh_ref, a2_ref):
        m = m_ref[...]
        m = jnp.where(jnp.isfinite(m), m, 0.0)
        onehot = oh_ref[...]
        mb = jnp.dot(onehot, m, preferred_element_type=jnp.float32)
        e = jnp.exp(s_ref[...] - mb)
        S = lax.dot_general(onehot, e, (((0,), (0,)), ((), ())),
                            preferred_element_type=jnp.float32)
        Sb = jnp.dot(onehot, S, preferred_element_type=jnp.float32)
        a2_ref[...] = e / (Sb + 1e-16)

    return pl.pallas_call(
        body,
        out_shape=jax.ShapeDtypeStruct((_N, 64), jnp.float32),
    )(s, m, onehot)


def _tc_outer(a2, x2, batch_col):
    def body(a2_ref, x2_ref, b_ref, t_ref):
        g = pl.program_id(0)
        mask = (b_ref[...] == g).astype(jnp.float32)
        Ag = a2_ref[...] * mask
        t_ref[0] = lax.dot_general(Ag, x2_ref[...], (((0,), (0,)), ((), ())),
                                   preferred_element_type=jnp.float32)

    return pl.pallas_call(
        body,
        grid=(_G,),
        in_specs=[
            pl.BlockSpec((_N, 64), lambda g: (0, 0)),
            pl.BlockSpec((_N, 64), lambda g: (0, 0)),
            pl.BlockSpec((_N, 1), lambda g: (0, 0)),
        ],
        out_specs=pl.BlockSpec((1, 64, 64), lambda g: (g, 0, 0)),
        out_shape=jax.ShapeDtypeStruct((_G, 64, 64), jnp.float32),
    )(a2, x2, batch_col)


def _tc_head(tflat, W_lin, b_lin):
    def body(t_ref, wl, bl, out_ref):
        logits = jnp.dot(t_ref[...], wl[...],
                         preferred_element_type=jnp.float32) + bl[...]
        lm = jnp.max(logits, axis=1, keepdims=True)
        le = jnp.exp(logits - lm)
        out_ref[...] = le / jnp.sum(le, axis=1, keepdims=True)

    return pl.pallas_call(
        body,
        out_shape=jax.ShapeDtypeStruct((_G, 10), jnp.float32),
    )(tflat, W_lin, b_lin)


def kernel(x, edge_index, batch, num_graphs, W_a1, b_a1, W_a2, b_a2,
           W_x1, b_x1, W_x2, b_x2, W_lin, b_lin):
    src = edge_index[0]
    dst = edge_index[1]
    npad_e = _EPAD - _E
    srcp = jnp.concatenate([src, jnp.zeros((npad_e,), jnp.int32)])
    dstp = jnp.concatenate([dst, jnp.full((npad_e,), _N, jnp.int32)])
    src2d = srcp.reshape(_EPAD // _CH, _CH)
    dst2d = dstp.reshape(_EPAD // _CH, _CH)

    zeros_hist = jnp.zeros((_NPAD,), jnp.float32)
    zeros_rows = jnp.zeros((_RPT, 128), jnp.float32)

    degf = _sc_degree(dstp, zeros_hist)
    deg0 = degf[:_N].reshape(_N, 1)
    deg1 = degf[_NPAD:_NPAD + _N].reshape(_N, 1)

    m0, dinv = _tc_prep(x, deg0, deg1)

    p = _sc_prop(m0, src2d, dst2d, zeros_rows)
    m1 = _tc_mid(p[:_N], p[_NPAD:_NPAD + _N], m0, dinv,
                 W_a1, b_a1.reshape(1, 128), W_x1, b_x1.reshape(1, 128),
                 W_a2, W_x2)

    q = _sc_prop(m1, src2d, dst2d, zeros_rows)
    batch_col = batch.reshape(_N, 1)
    s, x2, onehot = _tc_pre(q[:_N], q[_NPAD:_NPAD + _N], m1, dinv,
                            batch_col, b_a2.reshape(1, 64),
                            b_x2.reshape(1, 64))
    m = _tc_segmax(s, batch_col)
    a2 = _tc_softmax_norm(s, m, onehot)
    tall = _tc_outer(a2, x2, batch_col)
    out = _tc_head(tall.reshape(_G, 64 * 64), W_lin, b_lin.reshape(1, 10))
    return out


# speedup vs baseline: 14.8123x; 1.0250x over previous
import functools

import jax
import jax.numpy as jnp
from jax import lax
from jax.experimental import pallas as pl
from jax.experimental.pallas import tpu as pltpu
from jax.experimental.pallas import tpu_sc as plsc

_N = 10000
_E = 320000
_G = 64
_NC = 2
_NS = 16
_NW = _NC * _NS
_CH = 128
_NCHUNK = 80
_NCH0, _NCH1 = 144, 16
_SEG0, _SEG1 = 24, 8
_NSEG0, _NSEG1 = 6, 2
_EPT = _CH * _NCHUNK
_EPAD = _EPT * _NW
_NPAD = 10240
_RPT = _NPAD // _NS


def _sc_mesh():
    return plsc.VectorSubcoreMesh(core_axis_name="c", subcore_axis_name="s")


def _sc_degree(dstp, zeros_hist):
    @functools.partial(
        pl.kernel,
        out_type=jax.ShapeDtypeStruct((_NC * _NPAD,), jnp.float32),
        mesh=_sc_mesh(),
        scratch_types=[
            pltpu.VMEM((_EPT,), jnp.int32),
            pltpu.VMEM((_NPAD,), jnp.float32),
            pltpu.VMEM((_RPT,), jnp.float32),
            pltpu.VMEM((_RPT,), jnp.float32),
            pltpu.VMEM_SHARED((_NS, _NPAD), jnp.float32),
        ],
        compiler_params=pltpu.CompilerParams(needs_layout_passes=False),
    )
    def k(dst_hbm, zer_hbm, out_hbm, dix, hist, accb, tmpb, stage):
        c = lax.axis_index("c")
        s = lax.axis_index("s")
        w = c * _NS + s
        pltpu.sync_copy(zer_hbm, hist)
        pltpu.sync_copy(dst_hbm.at[pl.ds(w * _EPT, _EPT)], dix)
        ones = jnp.ones((16,), jnp.float32)

        def body(i, carry):
            idx = dix[pl.ds(i * 16, 16)]
            plsc.addupdate_scatter(hist, [idx], ones)
            return carry

        lax.fori_loop(0, _EPT // 16, body, 0)
        pltpu.sync_copy(hist, stage.at[s])
        plsc.subcore_barrier()

        pltpu.sync_copy(stage.at[0, pl.ds(s * _RPT, _RPT)], accb)
        for j in range(1, _NS):
            pltpu.sync_copy(stage.at[j, pl.ds(s * _RPT, _RPT)], tmpb)

            def addb(i, carry):
                sl = pl.ds(i * 16, 16)
                accb[sl] = accb[sl] + tmpb[sl]
                return carry

            lax.fori_loop(0, _RPT // 16, addb, 0)
        pltpu.sync_copy(accb, out_hbm.at[pl.ds(c * _NPAD + s * _RPT, _RPT)])

    return k(dstp, zeros_hist)


def _sc_prop(m, src2d, dst2d, zeros_rows):
    @functools.partial(
        pl.kernel,
        out_type=jax.ShapeDtypeStruct((_NC * _NPAD, 128), jnp.float32),
        mesh=_sc_mesh(),
        scratch_types=[
            pltpu.VMEM((_SEG0, _CH), jnp.int32),
            pltpu.VMEM((_SEG0, _CH), jnp.int32),
            pltpu.VMEM((_CH, 128), jnp.float32),
            pltpu.VMEM((_CH, 128), jnp.float32),
            pltpu.VMEM_SHARED((_NPAD, 128), jnp.float32),
            pltpu.SemaphoreType.DMA,
            pltpu.SemaphoreType.DMA,
        ],
    )
    def k(m_hbm, src_hbm, dst_hbm, zer_hbm, out_hbm,
          six2, dix2, rows0, rows1, acc, gsem, ssem):
        c = lax.axis_index("c")
        s = lax.axis_index("s")
        pltpu.sync_copy(zer_hbm, acc.at[pl.ds(s * _RPT, _RPT)])
        plsc.subcore_barrier()

        bufs = (rows0, rows1)

        def fire_gather(g, buf):
            pltpu.async_copy(m_hbm.at[six2.at[g]], buf, gsem)

        def drain_gather(buf):
            pltpu.make_async_copy(m_hbm.at[pl.ds(0, _CH)], buf, gsem).wait()

        def fire_scatter(g, buf):
            pltpu.async_copy(buf, acc.at[dix2.at[g]], ssem, add=True)

        def drain_scatter(buf):
            pltpu.make_async_copy(buf, acc.at[pl.ds(0, _CH)], ssem).wait()

        def run_seg(row0, nch):
            pltpu.sync_copy(src_hbm.at[pl.ds(row0, nch)],
                            six2.at[pl.ds(0, nch)])
            pltpu.sync_copy(dst_hbm.at[pl.ds(row0, nch)],
                            dix2.at[pl.ds(0, nch)])
            fire_gather(0, bufs[0])

            def body(tt, carry):
                for ph in (0, 1):
                    g = 2 * tt + ph
                    buf = bufs[ph]
                    drain_gather(buf)
                    fire_scatter(g, buf)

                    @pl.when(g >= 1)
                    def _():
                        drain_scatter(bufs[1 - ph])

                    @pl.when(g + 1 < nch)
                    def _():
                        fire_gather(g + 1, bufs[1 - ph])
                return carry

            lax.fori_loop(0, nch // 2, body, 0)
            drain_scatter(bufs[1])

        @pl.when(c == 0)
        def _():
            for seg in range(_NSEG0):
                run_seg(s * _NCH0 + seg * _SEG0, _SEG0)

        @pl.when(c == 1)
        def _():
            base = _NS * _NCH0
            for seg in range(_NSEG1):
                run_seg(base + s * _NCH1 + seg * _SEG1, _SEG1)

        plsc.subcore_barrier()
        pltpu.sync_copy(acc.at[pl.ds(s * _RPT, _RPT)],
                        out_hbm.at[pl.ds(c * _NPAD + s * _RPT, _RPT)])

    return k(m, src2d, dst2d, zeros_rows)


def _tc_prep(x, deg0, deg1):
    def body(x_ref, d0_ref, d1_ref, m0_ref, dinv_ref):
        deg = d0_ref[...] + d1_ref[...] + 1.0
        dinv = 1.0 / jnp.sqrt(deg)
        dinv_ref[...] = dinv
        m0_ref[...] = x_ref[...] * dinv

    return pl.pallas_call(
        body,
        out_shape=(
            jax.ShapeDtypeStruct((_N, 128), jnp.float32),
            jax.ShapeDtypeStruct((_N, 1), jnp.float32),
        ),
    )(x, deg0, deg1)


def _tc_mid(p0, p1, m0, dinv, W_a1, b_a1, W_x1, b_x1, W_a2, W_x2):
    def body(p0_ref, p1_ref, m0_ref, dinv_ref, wa1, ba1, wx1, bx1, wa2, wx2,
             m1_ref):
        P = (p0_ref[...] + p1_ref[...] + m0_ref[...]) * dinv_ref[...]
        a1 = jnp.maximum(
            jnp.dot(P, wa1[...], preferred_element_type=jnp.float32) + ba1[...],
            0.0)
        x1 = jnp.maximum(
            jnp.dot(P, wx1[...], preferred_element_type=jnp.float32) + bx1[...],
            0.0)
        ha = jnp.dot(a1, wa2[...], preferred_element_type=jnp.float32)
        hx = jnp.dot(x1, wx2[...], preferred_element_type=jnp.float32)
        m1_ref[...] = jnp.concatenate([ha, hx], axis=1) * dinv_ref[...]

    nb = 10
    rb = _N // nb
    return pl.pallas_call(
        body,
        grid=(nb,),
        in_specs=[
            pl.BlockSpec((rb, 128), lambda i: (i, 0)),
            pl.BlockSpec((rb, 128), lambda i: (i, 0)),
            pl.BlockSpec((rb, 128), lambda i: (i, 0)),
            pl.BlockSpec((rb, 1), lambda i: (i, 0)),
            pl.BlockSpec((128, 128), lambda i: (0, 0)),
            pl.BlockSpec((1, 128), lambda i: (0, 0)),
            pl.BlockSpec((128, 128), lambda i: (0, 0)),
            pl.BlockSpec((1, 128), lambda i: (0, 0)),
            pl.BlockSpec((128, 64), lambda i: (0, 0)),
            pl.BlockSpec((128, 64), lambda i: (0, 0)),
        ],
        out_specs=pl.BlockSpec((rb, 128), lambda i: (i, 0)),
        out_shape=jax.ShapeDtypeStruct((_N, 128), jnp.float32),
    )(p0, p1, m0, dinv, W_a1, b_a1, W_x1, b_x1, W_a2, W_x2)


def _tc_pre(q0, q1, m1, dinv, batch_col, b_a2, b_x2):
    def body(q0_ref, q1_ref, m1_ref, dinv_ref, b_ref, ba2, bx2,
             s_ref, x2_ref, oh_ref):
        q = (q0_ref[...] + q1_ref[...] + m1_ref[...]) * dinv_ref[...]
        s_ref[...] = q[:, :64] + ba2[...]
        x2_ref[...] = jnp.maximum(q[:, 64:] + bx2[...], 0.0)
        oh_ref[...] = (b_ref[...] == lax.broadcasted_iota(jnp.int32, (1, _G), 1)
                       ).astype(jnp.float32)

    return pl.pallas_call(
        body,
        out_shape=(
            jax.ShapeDtypeStruct((_N, 64), jnp.float32),
            jax.ShapeDtypeStruct((_N, 64), jnp.float32),
            jax.ShapeDtypeStruct((_N, _G), jnp.float32),
        ),
    )(q0, q1, m1, dinv, batch_col, b_a2, b_x2)


def _tc_segmax(s, batch_col):
    def body(s_ref, b_ref, m_ref):
        gb = pl.program_id(0) * 8
        b = b_ref[...]
        s = s_ref[...]
        rows = [jnp.max(jnp.where(b == gb + j, s, -jnp.inf), axis=0,
                        keepdims=True) for j in range(8)]
        m_ref[...] = jnp.concatenate(rows, axis=0)

    return pl.pallas_call(
        body,
        grid=(_G // 8,),
        in_specs=[
            pl.BlockSpec((_N, 64), lambda i: (0, 0)),
            pl.BlockSpec((_N, 1), lambda i: (0, 0)),
        ],
        out_specs=pl.BlockSpec((8, 64), lambda i: (i, 0)),
        out_shape=jax.ShapeDtypeStruct((_G, 64), jnp.float32),
    )(s, batch_col)


def _tc_softmax_norm(s, m, onehot):
    def body(s_ref, m_ref, oh_ref, a2_ref):
        m = m_ref[...]
        m = jnp.where(jnp.isfinite(m), m, 0.0)
        onehot = oh_ref[...]
        mb = jnp.dot(onehot, m, preferred_element_type=jnp.float32)
        e = jnp.exp(s_ref[...] - mb)
        S = lax.dot_general(onehot, e, (((0,), (0,)), ((), ())),
                            preferred_element_type=jnp.float32)
        Sb = jnp.dot(onehot, S, preferred_element_type=jnp.float32)
        a2_ref[...] = e / (Sb + 1e-16)

    return pl.pallas_call(
        body,
        out_shape=jax.ShapeDtypeStruct((_N, 64), jnp.float32),
    )(s, m, onehot)


def _tc_outer(a2, x2, batch_col):
    def body(a2_ref, x2_ref, b_ref, t_ref):
        g = pl.program_id(0)
        mask = (b_ref[...] == g).astype(jnp.float32)
        Ag = a2_ref[...] * mask
        t_ref[0] = lax.dot_general(Ag, x2_ref[...], (((0,), (0,)), ((), ())),
                                   preferred_element_type=jnp.float32)

    return pl.pallas_call(
        body,
        grid=(_G,),
        in_specs=[
            pl.BlockSpec((_N, 64), lambda g: (0, 0)),
            pl.BlockSpec((_N, 64), lambda g: (0, 0)),
            pl.BlockSpec((_N, 1), lambda g: (0, 0)),
        ],
        out_specs=pl.BlockSpec((1, 64, 64), lambda g: (g, 0, 0)),
        out_shape=jax.ShapeDtypeStruct((_G, 64, 64), jnp.float32),
    )(a2, x2, batch_col)


def _tc_head(tflat, W_lin, b_lin):
    def body(t_ref, wl, bl, out_ref):
        logits = jnp.dot(t_ref[...], wl[...],
                         preferred_element_type=jnp.float32) + bl[...]
        lm = jnp.max(logits, axis=1, keepdims=True)
        le = jnp.exp(logits - lm)
        out_ref[...] = le / jnp.sum(le, axis=1, keepdims=True)

    return pl.pallas_call(
        body,
        out_shape=jax.ShapeDtypeStruct((_G, 10), jnp.float32),
    )(tflat, W_lin, b_lin)


def kernel(x, edge_index, batch, num_graphs, W_a1, b_a1, W_a2, b_a2,
           W_x1, b_x1, W_x2, b_x2, W_lin, b_lin):
    src = edge_index[0]
    dst = edge_index[1]
    npad_e = _EPAD - _E
    srcp = jnp.concatenate([src, jnp.zeros((npad_e,), jnp.int32)])
    dstp = jnp.concatenate([dst, jnp.full((npad_e,), _N, jnp.int32)])
    src2d = srcp.reshape(_EPAD // _CH, _CH)
    dst2d = dstp.reshape(_EPAD // _CH, _CH)

    zeros_hist = jnp.zeros((_NPAD,), jnp.float32)
    zeros_rows = jnp.zeros((_RPT, 128), jnp.float32)

    degf = _sc_degree(dstp, zeros_hist)
    deg0 = degf[:_N].reshape(_N, 1)
    deg1 = degf[_NPAD:_NPAD + _N].reshape(_N, 1)

    m0, dinv = _tc_prep(x, deg0, deg1)

    p = _sc_prop(m0, src2d, dst2d, zeros_rows)
    m1 = _tc_mid(p[:_N], p[_NPAD:_NPAD + _N], m0, dinv,
                 W_a1, b_a1.reshape(1, 128), W_x1, b_x1.reshape(1, 128),
                 W_a2, W_x2)

    q = _sc_prop(m1, src2d, dst2d, zeros_rows)
    batch_col = batch.reshape(_N, 1)
    s, x2, onehot = _tc_pre(q[:_N], q[_NPAD:_NPAD + _N], m1, dinv,
                            batch_col, b_a2.reshape(1, 64),
                            b_x2.reshape(1, 64))
    m = _tc_segmax(s, batch_col)
    a2 = _tc_softmax_norm(s, m, onehot)
    tall = _tc_outer(a2, x2, batch_col)
    out = _tc_head(tall.reshape(_G, 64 * 64), W_lin, b_lin.reshape(1, 10))
    return out
